# baseline (speedup 1.0000x reference)
import functools

import jax
import jax.numpy as jnp
from jax import lax
from jax.experimental import pallas as pl
from jax.experimental.pallas import tpu as pltpu
from jax.experimental.pallas import tpu_sc as plsc

N = 10000
D = 128
E = 320000

NC = 2
NS = 16
CHUNK = 128
CPT0 = 38
CPT1 = 122
G = NS * (CPT0 + CPT1)
E_PAD = G * CHUNK
N_PAD = 10240
ROWS_PER_TILE = N_PAD // NS


def _scale_rows(rows, vals):
    def group_body(g2, carry):
        vv = vals[0, pl.ds(g2 * 16, 16)]
        for i in range(16):
            e = g2 * 16 + i
            v = vv[i]
            for g in range(D // 16):
                sl = pl.ds(g * 16, 16)
                rows[e, sl] = rows[e, sl] * v
        return carry
    lax.fori_loop(0, CHUNK // 16, group_body, 0)


def _spmm_body(x_hbm, pk_hbm, val_hbm, out_hbm,
               eb0, eb1, vb0, vb1, rows0, rows1,
               acc_sh,
               es0, es1, vs0, vs1, gs0, gs1):
    cid = lax.axis_index("c")
    sid = lax.axis_index("s")
    cpt = jnp.where(cid == 0, CPT0, CPT1)
    base = jnp.where(cid == 0, sid * CPT0, NS * CPT0 + sid * CPT1)

    def zrow(r, carry):
        for j in range(D // 16):
            rows0[r, pl.ds(j * 16, 16)] = jnp.zeros((16,), jnp.float32)
        return carry
    lax.fori_loop(0, CHUNK, zrow, 0)
    for k in range(ROWS_PER_TILE // CHUNK):
        pltpu.sync_copy(
            rows0, acc_sh.at[pl.ds(sid * ROWS_PER_TILE + k * CHUNK, CHUNK)])
    plsc.subcore_barrier()

    ebufs = (eb0, eb1)
    vbufs = (vb0, vb1)
    rbufs = (rows0, rows1)
    esems = (es0, es1)
    vsems = (vs0, vs1)
    gsems = (gs0, gs1)

    pltpu.async_copy(pk_hbm.at[base], eb0, es0).wait()
    pltpu.async_copy(val_hbm.at[base], vb0, vs0)
    pltpu.async_copy(x_hbm.at[eb0.at[1]], rows0, gs0)
    pltpu.async_copy(pk_hbm.at[base + 1], eb1, es1)
    pltpu.async_copy(val_hbm.at[base + 1], vb1, vs1)

    def step(c_dyn, p, q, issue):
        pltpu.make_async_copy(pk_hbm.at[base], ebufs[q], esems[q]).wait()
        pltpu.async_copy(x_hbm.at[ebufs[q].at[1]], rbufs[q], gsems[q])
        pltpu.make_async_copy(x_hbm.at[ebufs[p].at[1]], rbufs[p],
                              gsems[p]).wait()
        pltpu.make_async_copy(val_hbm.at[base], vbufs[p], vsems[p]).wait()
        _scale_rows(rbufs[p], vbufs[p])
        pltpu.sync_copy(rbufs[p], acc_sh.at[ebufs[p].at[0]], add=True)
        if issue:
            pltpu.async_copy(pk_hbm.at[c_dyn + 2], ebufs[p], esems[p])
            pltpu.async_copy(val_hbm.at[c_dyn + 2], vbufs[p], vsems[p])

    step(base, 0, 1, True)
    step(base + 1, 1, 0, True)

    def pair_body(t, carry):
        c = base + 2 * t
        step(c, 0, 1, True)
        step(c + 1, 1, 0, True)
        return carry
    lax.fori_loop(1, cpt // 2 - 1, pair_body, 0)

    step(base + cpt - 2, 0, 1, False)
    pltpu.make_async_copy(x_hbm.at[eb1.at[1]], rows1, gs1).wait()
    pltpu.make_async_copy(val_hbm.at[base], vb1, vs1).wait()
    _scale_rows(rows1, vb1)
    pltpu.sync_copy(rows1, acc_sh.at[eb1.at[0]], add=True)

    plsc.subcore_barrier()
    pltpu.sync_copy(
        acc_sh.at[pl.ds(sid * ROWS_PER_TILE, ROWS_PER_TILE)],
        out_hbm.at[cid, pl.ds(sid * ROWS_PER_TILE, ROWS_PER_TILE)])


_spmm = functools.partial(
    pl.kernel,
    mesh=plsc.VectorSubcoreMesh(core_axis_name="c", subcore_axis_name="s"),
    out_type=jax.ShapeDtypeStruct((NC, N_PAD, D), jnp.float32),
    scratch_types=[
        pltpu.VMEM((2, CHUNK), jnp.int32),
        pltpu.VMEM((2, CHUNK), jnp.int32),
        pltpu.VMEM((1, CHUNK), jnp.float32),
        pltpu.VMEM((1, CHUNK), jnp.float32),
        pltpu.VMEM((CHUNK, D), jnp.float32),
        pltpu.VMEM((CHUNK, D), jnp.float32),
        pltpu.VMEM_SHARED((N_PAD, D), jnp.float32),
        pltpu.SemaphoreType.DMA,
        pltpu.SemaphoreType.DMA,
        pltpu.SemaphoreType.DMA,
        pltpu.SemaphoreType.DMA,
        pltpu.SemaphoreType.DMA,
        pltpu.SemaphoreType.DMA,
    ],
)(_spmm_body)


BM = 400


def _mm_body(p_ref, w_ref, b_ref, o_ref):
    agg = p_ref[0] + p_ref[1]
    o_ref[...] = jnp.dot(agg, w_ref[...],
                         preferred_element_type=jnp.float32) + b_ref[...]


def _matmul(partials, W, b2):
    return pl.pallas_call(
        _mm_body,
        grid=(N // BM,),
        in_specs=[
            pl.BlockSpec((NC, BM, D), lambda i: (0, i, 0)),
            pl.BlockSpec((D, D), lambda i: (0, 0)),
            pl.BlockSpec((1, D), lambda i: (0, 0)),
        ],
        out_specs=pl.BlockSpec((BM, D), lambda i: (i, 0)),
        out_shape=jax.ShapeDtypeStruct((N, D), jnp.float32),
    )(partials, W, b2)


def kernel(x, edge_index, adj_values, W, b):
    row = edge_index[0]
    col = edge_index[1]
    pad = E_PAD - E
    zi = jnp.zeros((pad,), jnp.int32)
    row2 = jnp.concatenate([row, zi]).reshape(G, CHUNK)
    col2 = jnp.concatenate([col, zi]).reshape(G, CHUNK)
    packed = jnp.stack([row2, col2], axis=1)
    val3 = jnp.concatenate(
        [adj_values, jnp.zeros((pad,), jnp.float32)]).reshape(G, 1, CHUNK)
    partials = _spmm(x, packed, val3)
    return _matmul(partials, W, b.reshape(1, D))


# speedup vs baseline: 3.0983x; 3.0983x over previous
import functools

import jax
import jax.numpy as jnp
from jax import lax
from jax.experimental import pallas as pl
from jax.experimental.pallas import tpu as pltpu
from jax.experimental.pallas import tpu_sc as plsc

N = 10000
D = 128
E = 320000

NC = 2
NS = 16
CHUNK = 128
CPT0 = 80
CPT1 = 80
G = NS * (CPT0 + CPT1)
E_PAD = G * CHUNK
N_PAD = 10240
ROWS_PER_TILE = N_PAD // NS


def _scale_rows(rows, vals):
    def group_body(g2, carry):
        vv = vals[0, pl.ds(g2 * 16, 16)]
        for i in range(16):
            e = g2 * 16 + i
            v = vv[i]
            for g in range(D // 16):
                sl = pl.ds(g * 16, 16)
                rows[e, sl] = rows[e, sl] * v
        return carry
    lax.fori_loop(0, CHUNK // 16, group_body, 0)


def _spmm_body(x_hbm, pk_hbm, val_hbm, out_hbm,
               eb0, eb1, vb0, vb1, rows0, rows1,
               acc_sh,
               es0, es1, vs0, vs1, gs0, gs1):
    cid = lax.axis_index("c")
    sid = lax.axis_index("s")
    cpt = jnp.where(cid == 0, CPT0, CPT1)
    base = jnp.where(cid == 0, sid * CPT0, NS * CPT0 + sid * CPT1)

    def zrow(r, carry):
        for j in range(D // 16):
            rows0[r, pl.ds(j * 16, 16)] = jnp.zeros((16,), jnp.float32)
        return carry
    lax.fori_loop(0, CHUNK, zrow, 0)
    for k in range(ROWS_PER_TILE // CHUNK):
        pltpu.sync_copy(
            rows0, acc_sh.at[pl.ds(sid * ROWS_PER_TILE + k * CHUNK, CHUNK)])
    plsc.subcore_barrier()

    ebufs = (eb0, eb1)
    vbufs = (vb0, vb1)
    rbufs = (rows0, rows1)
    esems = (es0, es1)
    vsems = (vs0, vs1)
    gsems = (gs0, gs1)

    pltpu.async_copy(pk_hbm.at[base], eb0, es0).wait()
    pltpu.async_copy(val_hbm.at[base], vb0, vs0)
    pltpu.async_copy(x_hbm.at[eb0.at[1]], rows0, gs0)
    pltpu.async_copy(pk_hbm.at[base + 1], eb1, es1)
    pltpu.async_copy(val_hbm.at[base + 1], vb1, vs1)

    def step(c_dyn, p, q, issue):
        pltpu.make_async_copy(pk_hbm.at[base], ebufs[q], esems[q]).wait()
        pltpu.async_copy(x_hbm.at[ebufs[q].at[1]], rbufs[q], gsems[q])
        pltpu.make_async_copy(x_hbm.at[ebufs[p].at[1]], rbufs[p],
                              gsems[p]).wait()
        pltpu.make_async_copy(val_hbm.at[base], vbufs[p], vsems[p]).wait()
        _scale_rows(rbufs[p], vbufs[p])
        pltpu.sync_copy(rbufs[p], acc_sh.at[ebufs[p].at[0]], add=True)
        if issue:
            pltpu.async_copy(pk_hbm.at[c_dyn + 2], ebufs[p], esems[p])
            pltpu.async_copy(val_hbm.at[c_dyn + 2], vbufs[p], vsems[p])

    step(base, 0, 1, True)
    step(base + 1, 1, 0, True)

    def pair_body(t, carry):
        c = base + 2 * t
        step(c, 0, 1, True)
        step(c + 1, 1, 0, True)
        return carry
    lax.fori_loop(1, cpt // 2 - 1, pair_body, 0)

    step(base + cpt - 2, 0, 1, False)
    pltpu.make_async_copy(x_hbm.at[eb1.at[1]], rows1, gs1).wait()
    pltpu.make_async_copy(val_hbm.at[base], vb1, vs1).wait()
    _scale_rows(rows1, vb1)
    pltpu.sync_copy(rows1, acc_sh.at[eb1.at[0]], add=True)

    plsc.subcore_barrier()
    pltpu.sync_copy(
        acc_sh.at[pl.ds(sid * ROWS_PER_TILE, ROWS_PER_TILE)],
        out_hbm.at[cid, pl.ds(sid * ROWS_PER_TILE, ROWS_PER_TILE)])


_spmm = functools.partial(
    pl.kernel,
    mesh=plsc.VectorSubcoreMesh(core_axis_name="c", subcore_axis_name="s"),
    out_type=jax.ShapeDtypeStruct((NC, N_PAD, D), jnp.float32),
    scratch_types=[
        pltpu.VMEM((2, CHUNK), jnp.int32),
        pltpu.VMEM((2, CHUNK), jnp.int32),
        pltpu.VMEM((1, CHUNK), jnp.float32),
        pltpu.VMEM((1, CHUNK), jnp.float32),
        pltpu.VMEM((CHUNK, D), jnp.float32),
        pltpu.VMEM((CHUNK, D), jnp.float32),
        pltpu.VMEM_SHARED((N_PAD, D), jnp.float32),
        pltpu.SemaphoreType.DMA,
        pltpu.SemaphoreType.DMA,
        pltpu.SemaphoreType.DMA,
        pltpu.SemaphoreType.DMA,
        pltpu.SemaphoreType.DMA,
        pltpu.SemaphoreType.DMA,
    ],
)(_spmm_body)


BM = 400


def _mm_body(p_ref, w_ref, b_ref, o_ref):
    agg = p_ref[0] + p_ref[1]
    o_ref[...] = jnp.dot(agg, w_ref[...],
                         preferred_element_type=jnp.float32) + b_ref[...]


def _matmul(partials, W, b2):
    return pl.pallas_call(
        _mm_body,
        grid=(N // BM,),
        in_specs=[
            pl.BlockSpec((NC, BM, D), lambda i: (0, i, 0)),
            pl.BlockSpec((D, D), lambda i: (0, 0)),
            pl.BlockSpec((1, D), lambda i: (0, 0)),
        ],
        out_specs=pl.BlockSpec((BM, D), lambda i: (i, 0)),
        out_shape=jax.ShapeDtypeStruct((N, D), jnp.float32),
    )(partials, W, b2)


def kernel(x, edge_index, adj_values, W, b):
    row = edge_index[0]
    col = edge_index[1]
    pad = E_PAD - E
    pr = jnp.arange(pad, dtype=jnp.int32) % N_PAD
    pc = jnp.arange(pad, dtype=jnp.int32) % N
    row2 = jnp.concatenate([row, pr]).reshape(G, CHUNK)
    col2 = jnp.concatenate([col, pc]).reshape(G, CHUNK)
    packed = jnp.stack([row2, col2], axis=1)
    val3 = jnp.concatenate(
        [adj_values, jnp.zeros((pad,), jnp.float32)]).reshape(G, 1, CHUNK)
    partials = _spmm(x, packed, val3)
    return _matmul(partials, W, b.reshape(1, D))
